# Initial kernel scaffold; baseline (speedup 1.0000x reference)
import jax
import jax.numpy as jnp
from jax.experimental import pallas as pl


def kernel(x, edge_index, W_rel1, b_rel1, W_root1, gamma1, beta1, W_rel2, b_rel2, W_root2, gamma2, beta2, W_rel3, b_rel3, W_root3):
    raise NotImplementedError("write your pallas kernel here")


# speedup vs baseline: 6.0694x; 6.0694x over previous
import functools

import jax
import jax.numpy as jnp
from jax import lax
from jax.experimental import pallas as pl
from jax.experimental.pallas import tpu as pltpu
from jax.experimental.pallas import tpu_sc as plsc

_N = 10000
_E = 320000
_D_IN = 128
_D_H = 128
_D_OUT = 64
_EPS = 1e-5

_NP = 10240
_NC = 2
_NS = 16
_ROWS_PER_TILE = _NP // _NS
_EDGES_PER_SC = _E // _NC
_EDGES_PER_TILE = _EDGES_PER_SC // _NS
_CH = 80
_NCHUNK = _EDGES_PER_TILE // _CH


@functools.partial(jax.jit, static_argnames=("d",))
def _sc_segment_sum(g, src, dst, zeros, d):
    mesh = plsc.VectorSubcoreMesh(core_axis_name="c", subcore_axis_name="s")

    @functools.partial(
        pl.kernel,
        out_type=jax.ShapeDtypeStruct((_NC * _NP, d), jnp.float32),
        mesh=mesh,
        scratch_types=[
            pltpu.VMEM((_NCHUNK, _CH), jnp.int32),
            pltpu.VMEM((_NCHUNK, _CH), jnp.int32),
            pltpu.VMEM((_CH, d), jnp.float32),
            pltpu.VMEM_SHARED((_NP, d), jnp.float32),
        ],
    )
    def seg(g_hbm, src_hbm, dst_hbm, z_hbm, out_hbm, src_v, dst_v, rows_v, acc):
        c = lax.axis_index("c")
        s = lax.axis_index("s")
        row0 = s * _ROWS_PER_TILE
        pltpu.sync_copy(z_hbm.at[pl.ds(row0, _ROWS_PER_TILE)],
                        acc.at[pl.ds(row0, _ROWS_PER_TILE)])
        pltpu.sync_copy(src_hbm.at[c, s], src_v)
        pltpu.sync_copy(dst_hbm.at[c, s], dst_v)
        plsc.subcore_barrier()

        def chunk(i, carry):
            pltpu.sync_copy(g_hbm.at[src_v.at[i]], rows_v)
            pltpu.sync_copy(rows_v, acc.at[dst_v.at[i]], add=True)
            return carry

        lax.fori_loop(0, _NCHUNK, chunk, 0)
        plsc.subcore_barrier()
        pltpu.sync_copy(acc.at[pl.ds(row0, _ROWS_PER_TILE)],
                        out_hbm.at[pl.ds(c * _NP + row0, _ROWS_PER_TILE)])

    return seg(g, src, dst, zeros)


def _dot(a, b):
    return jnp.dot(a, b, preferred_element_type=jnp.float32,
                   precision=lax.Precision.HIGHEST)


def _tc_pre_body(x_ref, w_ref, g_ref):
    g_ref[...] = _dot(x_ref[...], w_ref[...])


def _tc_mid_body(msgp_ref, h_ref, wroot_ref, b_ref, gam_ref, bet_ref,
                 wnext_ref, h_out_ref, g_out_ref):
    msg = msgp_ref[pl.ds(0, _NP), :] + msgp_ref[pl.ds(_NP, _NP), :]
    u = msg + _dot(h_ref[...], wroot_ref[...]) + b_ref[...]
    uv = u[:_N, :]
    mu = jnp.mean(uv, axis=0, keepdims=True)
    var = jnp.mean((uv - mu) ** 2, axis=0, keepdims=True)
    hn = (u - mu) * lax.rsqrt(var + _EPS) * gam_ref[...] + bet_ref[...]
    h1 = jnp.maximum(hn, 0.0)
    h_out_ref[...] = h1
    g_out_ref[...] = _dot(h1, wnext_ref[...])


def _tc_mid2_body(msgp_ref, h_ref, wroot_ref, b_ref, gam_ref, bet_ref,
                  h_out_ref):
    msg = msgp_ref[pl.ds(0, _NP), :] + msgp_ref[pl.ds(_NP, _NP), :]
    u = msg + _dot(h_ref[...], wroot_ref[...]) + b_ref[...]
    uv = u[:_N, :]
    mu = jnp.mean(uv, axis=0, keepdims=True)
    var = jnp.mean((uv - mu) ** 2, axis=0, keepdims=True)
    hn = (u - mu) * lax.rsqrt(var + _EPS) * gam_ref[...] + bet_ref[...]
    h_out_ref[...] = jnp.maximum(hn, 0.0)


def _tc_final_body(msgp_ref, h_ref, wrel_ref, wroot_ref, b_ref, out_ref):
    msg = msgp_ref[pl.ds(0, _N), :] + msgp_ref[pl.ds(_NP, _N), :]
    u = (_dot(msg, wrel_ref[...]) +
         _dot(h_ref[pl.ds(0, _N), :], wroot_ref[...]) + b_ref[...])
    m = jnp.max(u, axis=-1, keepdims=True)
    lse = jnp.log(jnp.sum(jnp.exp(u - m), axis=-1, keepdims=True)) + m
    out_ref[...] = u - lse


def _tc_pre(x, w_t, d_out):
    return pl.pallas_call(
        _tc_pre_body,
        out_shape=jax.ShapeDtypeStruct((_NP, d_out), jnp.float32),
    )(x, w_t)


def _tc_mid(msgp, h, wroot_t, b, gam, bet, wnext_t, d_next):
    return pl.pallas_call(
        _tc_mid_body,
        out_shape=(jax.ShapeDtypeStruct((_NP, _D_H), jnp.float32),
                   jax.ShapeDtypeStruct((_NP, d_next), jnp.float32)),
    )(msgp, h, wroot_t, b, gam, bet, wnext_t)


def _tc_mid2(msgp, h, wroot_t, b, gam, bet):
    return pl.pallas_call(
        _tc_mid2_body,
        out_shape=jax.ShapeDtypeStruct((_NP, _D_H), jnp.float32),
    )(msgp, h, wroot_t, b, gam, bet)


def _tc_final(msgp, h, wrel_t, wroot_t, b):
    return pl.pallas_call(
        _tc_final_body,
        out_shape=jax.ShapeDtypeStruct((_N, _D_OUT), jnp.float32),
    )(msgp, h, wrel_t, wroot_t, b)


def kernel(x, edge_index, W_rel1, b_rel1, W_root1, gamma1, beta1,
           W_rel2, b_rel2, W_root2, gamma2, beta2,
           W_rel3, b_rel3, W_root3):
    src = edge_index[0].reshape(_NC, _NS, _NCHUNK, _CH)
    dst = edge_index[1].reshape(_NC, _NS, _NCHUNK, _CH)
    zeros_h = jnp.zeros((_NP, _D_H), jnp.float32)
    xp = jnp.pad(x, ((0, _NP - _N), (0, 0)))

    g1 = _tc_pre(xp, W_rel1.T, _D_H)
    m1 = _sc_segment_sum(g1, src, dst, zeros_h, d=_D_H)
    h1, g2 = _tc_mid(m1, xp, W_root1.T, b_rel1.reshape(1, -1),
                     gamma1.reshape(1, -1), beta1.reshape(1, -1),
                     W_rel2.T, _D_H)
    m2 = _sc_segment_sum(g2, src, dst, zeros_h, d=_D_H)
    h2 = _tc_mid2(m2, h1, W_root2.T, b_rel2.reshape(1, -1),
                  gamma2.reshape(1, -1), beta2.reshape(1, -1))
    m3 = _sc_segment_sum(h2, src, dst, zeros_h, d=_D_H)
    return _tc_final(m3, h2, W_rel3.T, W_root3.T, b_rel3.reshape(1, -1))


# speedup vs baseline: 8.5566x; 1.4098x over previous
import functools

import jax
import jax.numpy as jnp
from jax import lax
from jax.experimental import pallas as pl
from jax.experimental.pallas import tpu as pltpu
from jax.experimental.pallas import tpu_sc as plsc

_N = 10000
_E = 320000
_D_IN = 128
_D_H = 128
_D_OUT = 64
_EPS = 1e-5

_NP = 10240
_NC = 2
_NS = 16
_ROWS_PER_TILE = _NP // _NS
_EDGES_PER_SC = _E // _NC
_EDGES_PER_TILE = _EDGES_PER_SC // _NS
_CH = 80
_NCHUNK = _EDGES_PER_TILE // _CH


@functools.partial(jax.jit, static_argnames=("d",))
def _sc_segment_sum(g, src, dst, zeros, d):
    mesh = plsc.VectorSubcoreMesh(core_axis_name="c", subcore_axis_name="s")

    @functools.partial(
        pl.kernel,
        out_type=jax.ShapeDtypeStruct((_NC * _NP, d), jnp.float32),
        mesh=mesh,
        scratch_types=[
            pltpu.VMEM((_EDGES_PER_TILE,), jnp.int32),
            pltpu.VMEM((_NCHUNK, _CH), jnp.int32),
            pltpu.VMEM((_CH, d), jnp.float32),
            pltpu.VMEM((_CH, d), jnp.float32),
            pltpu.VMEM_SHARED((_NP, d), jnp.float32),
            pltpu.SemaphoreType.DMA,
            pltpu.SemaphoreType.DMA,
            pltpu.SemaphoreType.DMA,
            pltpu.SemaphoreType.DMA,
        ],
    )
    def seg(g_hbm, src_hbm, dst_hbm, z_hbm, out_hbm, src_v, dst_v,
            rows_a, rows_b, acc, sem_ga, sem_gb, sem_sa, sem_sb):
        c = lax.axis_index("c")
        s = lax.axis_index("s")
        row0 = s * _ROWS_PER_TILE
        pltpu.sync_copy(z_hbm, rows_a)
        nz = _ROWS_PER_TILE // _CH

        def zissue(j, carry):
            pltpu.async_copy(rows_a, acc.at[pl.ds(row0 + j * _CH, _CH)],
                             sem_sa)
            return carry

        lax.fori_loop(0, nz, zissue, 0)
        pltpu.async_copy(src_hbm.at[c, s], src_v, sem_ga)
        pltpu.async_copy(dst_hbm.at[c, s], dst_v, sem_gb)

        def zdrain(j, carry):
            pltpu.make_async_copy(rows_a, acc.at[pl.ds(row0, _CH)],
                                  sem_sa).wait()
            return carry

        lax.fori_loop(0, nz, zdrain, 0)
        pltpu.make_async_copy(src_hbm.at[c, s], src_v, sem_ga).wait()
        pltpu.make_async_copy(dst_hbm.at[c, s], dst_v, sem_gb).wait()
        plsc.subcore_barrier()

        pltpu.async_copy(g_hbm.at[src_v.at[pl.ds(0, _CH)]], rows_a, sem_ga)
        pltpu.async_copy(g_hbm.at[src_v.at[pl.ds(_CH, _CH)]], rows_b, sem_gb)

        def pair(k, carry):
            i0 = 2 * k
            i1 = i0 + 1
            pltpu.make_async_copy(g_hbm.at[src_v.at[pl.ds(i0 * _CH, _CH)]], rows_a, sem_ga).wait()
            pltpu.async_copy(rows_a, acc.at[dst_v.at[i0]], sem_sa, add=True)
            pltpu.make_async_copy(g_hbm.at[src_v.at[pl.ds(i1 * _CH, _CH)]], rows_b, sem_gb).wait()
            pltpu.async_copy(rows_b, acc.at[dst_v.at[i1]], sem_sb, add=True)
            pltpu.make_async_copy(rows_a, acc.at[dst_v.at[i0]], sem_sa).wait()
            pltpu.async_copy(g_hbm.at[src_v.at[pl.ds((i0 + 2) * _CH, _CH)]], rows_a, sem_ga)
            ib = jnp.minimum(i1 + 2, _NCHUNK - 1)
            pltpu.make_async_copy(rows_b, acc.at[dst_v.at[i1]], sem_sb).wait()
            pltpu.async_copy(g_hbm.at[src_v.at[pl.ds(ib * _CH, _CH)]], rows_b, sem_gb)
            return carry

        lax.fori_loop(0, (_NCHUNK - 1) // 2, pair, 0)
        last = _NCHUNK - 1
        pltpu.make_async_copy(g_hbm.at[src_v.at[pl.ds(last * _CH, _CH)]], rows_a, sem_ga).wait()
        pltpu.sync_copy(rows_a, acc.at[dst_v.at[last]], add=True)
        pltpu.make_async_copy(g_hbm.at[src_v.at[pl.ds(last * _CH, _CH)]], rows_b, sem_gb).wait()
        plsc.subcore_barrier()

        r0a = row0
        r0b = row0 + _CH
        pltpu.sync_copy(acc.at[pl.ds(r0a, _CH)], rows_a)
        pltpu.async_copy(rows_a, out_hbm.at[pl.ds(c * _NP + r0a, _CH)], sem_sa)
        pltpu.sync_copy(acc.at[pl.ds(r0b, _CH)], rows_b)
        pltpu.async_copy(rows_b, out_hbm.at[pl.ds(c * _NP + r0b, _CH)], sem_sb)

        def wpair(j, carry):
            ra = row0 + (2 * j + 2) * _CH
            rb = row0 + (2 * j + 3) * _CH
            pltpu.make_async_copy(
                rows_a, out_hbm.at[pl.ds(c * _NP + ra, _CH)], sem_sa).wait()
            pltpu.sync_copy(acc.at[pl.ds(ra, _CH)], rows_a)
            pltpu.async_copy(rows_a, out_hbm.at[pl.ds(c * _NP + ra, _CH)],
                             sem_sa)
            pltpu.make_async_copy(
                rows_b, out_hbm.at[pl.ds(c * _NP + rb, _CH)], sem_sb).wait()
            pltpu.sync_copy(acc.at[pl.ds(rb, _CH)], rows_b)
            pltpu.async_copy(rows_b, out_hbm.at[pl.ds(c * _NP + rb, _CH)],
                             sem_sb)
            return carry

        lax.fori_loop(0, _ROWS_PER_TILE // (2 * _CH) - 1, wpair, 0)
        pltpu.make_async_copy(
            rows_a, out_hbm.at[pl.ds(c * _NP + row0, _CH)], sem_sa).wait()
        pltpu.make_async_copy(
            rows_b, out_hbm.at[pl.ds(c * _NP + row0, _CH)], sem_sb).wait()

    return seg(g, src, dst, zeros)


def _dot(a, b):
    return jnp.dot(a, b, preferred_element_type=jnp.float32)


def _tc_pre_body(x_ref, w_ref, g_ref):
    g_ref[...] = _dot(x_ref[...], w_ref[...])


def _tc_mid_body(msgp_ref, h_ref, wroot_ref, b_ref, gam_ref, bet_ref,
                 wnext_ref, h_out_ref, g_out_ref):
    msg = msgp_ref[pl.ds(0, _N), :] + msgp_ref[pl.ds(_NP, _N), :]
    u = msg + _dot(h_ref[...], wroot_ref[...]) + b_ref[...]
    mu = jnp.mean(u, axis=0, keepdims=True)
    var = jnp.mean((u - mu) ** 2, axis=0, keepdims=True)
    hn = (u - mu) * lax.rsqrt(var + _EPS) * gam_ref[...] + bet_ref[...]
    h1 = jnp.maximum(hn, 0.0)
    h_out_ref[...] = h1
    g_out_ref[...] = _dot(h1, wnext_ref[...])


def _tc_mid2_body(msgp_ref, h_ref, wroot_ref, b_ref, gam_ref, bet_ref,
                  h_out_ref):
    msg = msgp_ref[pl.ds(0, _N), :] + msgp_ref[pl.ds(_NP, _N), :]
    u = msg + _dot(h_ref[...], wroot_ref[...]) + b_ref[...]
    mu = jnp.mean(u, axis=0, keepdims=True)
    var = jnp.mean((u - mu) ** 2, axis=0, keepdims=True)
    hn = (u - mu) * lax.rsqrt(var + _EPS) * gam_ref[...] + bet_ref[...]
    h_out_ref[...] = jnp.maximum(hn, 0.0)


def _tc_final_body(msgp_ref, h_ref, wrel_ref, wroot_ref, b_ref, out_ref):
    msg = msgp_ref[pl.ds(0, _N), :] + msgp_ref[pl.ds(_NP, _N), :]
    u = (_dot(msg, wrel_ref[...]) +
         _dot(h_ref[...], wroot_ref[...]) + b_ref[...])
    m = jnp.max(u, axis=-1, keepdims=True)
    lse = jnp.log(jnp.sum(jnp.exp(u - m), axis=-1, keepdims=True)) + m
    out_ref[...] = u - lse


def _tc_pre(x, w_t, d_out):
    return pl.pallas_call(
        _tc_pre_body,
        out_shape=jax.ShapeDtypeStruct((_N, d_out), jnp.float32),
    )(x, w_t)


def _tc_mid(msgp, h, wroot_t, b, gam, bet, wnext_t, d_next):
    return pl.pallas_call(
        _tc_mid_body,
        out_shape=(jax.ShapeDtypeStruct((_N, _D_H), jnp.float32),
                   jax.ShapeDtypeStruct((_N, d_next), jnp.float32)),
    )(msgp, h, wroot_t, b, gam, bet, wnext_t)


def _tc_mid2(msgp, h, wroot_t, b, gam, bet):
    return pl.pallas_call(
        _tc_mid2_body,
        out_shape=jax.ShapeDtypeStruct((_N, _D_H), jnp.float32),
    )(msgp, h, wroot_t, b, gam, bet)


def _tc_final(msgp, h, wrel_t, wroot_t, b):
    return pl.pallas_call(
        _tc_final_body,
        out_shape=jax.ShapeDtypeStruct((_N, _D_OUT), jnp.float32),
    )(msgp, h, wrel_t, wroot_t, b)


def kernel(x, edge_index, W_rel1, b_rel1, W_root1, gamma1, beta1,
           W_rel2, b_rel2, W_root2, gamma2, beta2,
           W_rel3, b_rel3, W_root3):
    src = edge_index[0].reshape(_NC, _NS, _EDGES_PER_TILE)
    dst = edge_index[1].reshape(_NC, _NS, _NCHUNK, _CH)
    zeros_h = jnp.zeros((_CH, _D_H), jnp.float32)

    g1 = _tc_pre(x, W_rel1.T, _D_H)
    m1 = _sc_segment_sum(g1, src, dst, zeros_h, d=_D_H)
    h1, g2 = _tc_mid(m1, x, W_root1.T, b_rel1.reshape(1, -1),
                     gamma1.reshape(1, -1), beta1.reshape(1, -1),
                     W_rel2.T, _D_H)
    m2 = _sc_segment_sum(g2, src, dst, zeros_h, d=_D_H)
    h2 = _tc_mid2(m2, h1, W_root2.T, b_rel2.reshape(1, -1),
                  gamma2.reshape(1, -1), beta2.reshape(1, -1))
    m3 = _sc_segment_sum(h2, src, dst, zeros_h, d=_D_H)
    return _tc_final(m3, h2, W_rel3.T, W_root3.T, b_rel3.reshape(1, -1))


# speedup vs baseline: 8.7743x; 1.0254x over previous
import functools

import jax
import jax.numpy as jnp
from jax import lax
from jax.experimental import pallas as pl
from jax.experimental.pallas import tpu as pltpu
from jax.experimental.pallas import tpu_sc as plsc

_N = 10000
_E = 320000
_D_IN = 128
_D_H = 128
_D_OUT = 64
_EPS = 1e-5

_NP = 10240
_NC = 2
_NS = 16
_ROWS_PER_TILE = _NP // _NS
_EDGES_PER_SC = _E // _NC
_EDGES_PER_TILE = _EDGES_PER_SC // _NS
_CH = 80
_NCHUNK = _EDGES_PER_TILE // _CH


@functools.partial(jax.jit, static_argnames=("d",))
def _sc_segment_sum(g, src, dst, zeros, d):
    mesh = plsc.VectorSubcoreMesh(core_axis_name="c", subcore_axis_name="s")

    @functools.partial(
        pl.kernel,
        out_type=jax.ShapeDtypeStruct((_NC * _NP, d), jnp.float32),
        mesh=mesh,
        scratch_types=[
            pltpu.VMEM((_EDGES_PER_TILE,), jnp.int32),
            pltpu.VMEM((_NCHUNK, _CH), jnp.int32),
            pltpu.VMEM((_CH, d), jnp.float32),
            pltpu.VMEM((_CH, d), jnp.float32),
            pltpu.VMEM_SHARED((_NP, d), jnp.float32),
            pltpu.SemaphoreType.DMA,
            pltpu.SemaphoreType.DMA,
            pltpu.SemaphoreType.DMA,
            pltpu.SemaphoreType.DMA,
            pltpu.SemaphoreType.DMA,
            pltpu.SemaphoreType.DMA,
        ],
    )
    def seg(g_hbm, src_hbm, dst_hbm, z_hbm, out_hbm, src_v, dst_v,
            rows_a, rows_b, acc, sem_ga, sem_gb, sem_sa, sem_sb,
            sem_ga2, sem_gb2):
        c = lax.axis_index("c")
        s = lax.axis_index("s")
        row0 = s * _ROWS_PER_TILE
        pltpu.sync_copy(z_hbm, rows_a)
        nz = _ROWS_PER_TILE // _CH

        def zissue(j, carry):
            pltpu.async_copy(rows_a, acc.at[pl.ds(row0 + j * _CH, _CH)],
                             sem_sa)
            return carry

        lax.fori_loop(0, nz, zissue, 0)
        pltpu.async_copy(src_hbm.at[c, s], src_v, sem_ga)
        pltpu.async_copy(dst_hbm.at[c, s], dst_v, sem_gb)

        def zdrain(j, carry):
            pltpu.make_async_copy(rows_a, acc.at[pl.ds(row0, _CH)],
                                  sem_sa).wait()
            return carry

        lax.fori_loop(0, nz, zdrain, 0)
        pltpu.make_async_copy(src_hbm.at[c, s], src_v, sem_ga).wait()
        pltpu.make_async_copy(dst_hbm.at[c, s], dst_v, sem_gb).wait()
        plsc.subcore_barrier()

        _H = _CH // 2

        def issue_gather(i, buf, sem_lo, sem_hi):
            pltpu.async_copy(g_hbm.at[src_v.at[pl.ds(i * _CH, _H)]],
                             buf.at[pl.ds(0, _H)], sem_lo)
            pltpu.async_copy(g_hbm.at[src_v.at[pl.ds(i * _CH + _H, _H)]],
                             buf.at[pl.ds(_H, _H)], sem_hi)

        def wait_gather(i, buf, sem_lo, sem_hi):
            pltpu.make_async_copy(g_hbm.at[src_v.at[pl.ds(i * _CH, _H)]],
                                  buf.at[pl.ds(0, _H)], sem_lo).wait()
            pltpu.make_async_copy(g_hbm.at[src_v.at[pl.ds(i * _CH, _H)]],
                                  buf.at[pl.ds(_H, _H)], sem_hi).wait()

        issue_gather(0, rows_a, sem_ga, sem_ga2)
        issue_gather(1, rows_b, sem_gb, sem_gb2)

        def pair(k, carry):
            i0 = 2 * k
            i1 = i0 + 1
            wait_gather(i0, rows_a, sem_ga, sem_ga2)
            pltpu.async_copy(rows_a, acc.at[dst_v.at[i0]], sem_sa, add=True)
            wait_gather(i1, rows_b, sem_gb, sem_gb2)
            pltpu.async_copy(rows_b, acc.at[dst_v.at[i1]], sem_sb, add=True)
            pltpu.make_async_copy(rows_a, acc.at[dst_v.at[i0]], sem_sa).wait()
            issue_gather(i0 + 2, rows_a, sem_ga, sem_ga2)
            ib = jnp.minimum(i1 + 2, _NCHUNK - 1)
            pltpu.make_async_copy(rows_b, acc.at[dst_v.at[i1]], sem_sb).wait()
            issue_gather(ib, rows_b, sem_gb, sem_gb2)
            return carry

        lax.fori_loop(0, (_NCHUNK - 1) // 2, pair, 0)
        last = _NCHUNK - 1
        wait_gather(last, rows_a, sem_ga, sem_ga2)
        pltpu.sync_copy(rows_a, acc.at[dst_v.at[last]], add=True)
        wait_gather(last, rows_b, sem_gb, sem_gb2)
        plsc.subcore_barrier()

        r0a = row0
        r0b = row0 + _CH
        pltpu.sync_copy(acc.at[pl.ds(r0a, _CH)], rows_a)
        pltpu.async_copy(rows_a, out_hbm.at[pl.ds(c * _NP + r0a, _CH)], sem_sa)
        pltpu.sync_copy(acc.at[pl.ds(r0b, _CH)], rows_b)
        pltpu.async_copy(rows_b, out_hbm.at[pl.ds(c * _NP + r0b, _CH)], sem_sb)

        def wpair(j, carry):
            ra = row0 + (2 * j + 2) * _CH
            rb = row0 + (2 * j + 3) * _CH
            pltpu.make_async_copy(
                rows_a, out_hbm.at[pl.ds(c * _NP + ra, _CH)], sem_sa).wait()
            pltpu.sync_copy(acc.at[pl.ds(ra, _CH)], rows_a)
            pltpu.async_copy(rows_a, out_hbm.at[pl.ds(c * _NP + ra, _CH)],
                             sem_sa)
            pltpu.make_async_copy(
                rows_b, out_hbm.at[pl.ds(c * _NP + rb, _CH)], sem_sb).wait()
            pltpu.sync_copy(acc.at[pl.ds(rb, _CH)], rows_b)
            pltpu.async_copy(rows_b, out_hbm.at[pl.ds(c * _NP + rb, _CH)],
                             sem_sb)
            return carry

        lax.fori_loop(0, _ROWS_PER_TILE // (2 * _CH) - 1, wpair, 0)
        pltpu.make_async_copy(
            rows_a, out_hbm.at[pl.ds(c * _NP + row0, _CH)], sem_sa).wait()
        pltpu.make_async_copy(
            rows_b, out_hbm.at[pl.ds(c * _NP + row0, _CH)], sem_sb).wait()

    return seg(g, src, dst, zeros)


def _dot(a, b):
    return jnp.dot(a, b, preferred_element_type=jnp.float32)


def _tc_pre_body(x_ref, w_ref, g_ref):
    g_ref[...] = _dot(x_ref[...], w_ref[...])


def _tc_mid_body(msgp_ref, h_ref, wroot_ref, b_ref, gam_ref, bet_ref,
                 wnext_ref, h_out_ref, g_out_ref):
    msg = msgp_ref[pl.ds(0, _N), :] + msgp_ref[pl.ds(_NP, _N), :]
    u = msg + _dot(h_ref[...], wroot_ref[...]) + b_ref[...]
    mu = jnp.mean(u, axis=0, keepdims=True)
    var = jnp.mean((u - mu) ** 2, axis=0, keepdims=True)
    hn = (u - mu) * lax.rsqrt(var + _EPS) * gam_ref[...] + bet_ref[...]
    h1 = jnp.maximum(hn, 0.0)
    h_out_ref[...] = h1
    g_out_ref[...] = _dot(h1, wnext_ref[...])


def _tc_mid2_body(msgp_ref, h_ref, wroot_ref, b_ref, gam_ref, bet_ref,
                  h_out_ref):
    msg = msgp_ref[pl.ds(0, _N), :] + msgp_ref[pl.ds(_NP, _N), :]
    u = msg + _dot(h_ref[...], wroot_ref[...]) + b_ref[...]
    mu = jnp.mean(u, axis=0, keepdims=True)
    var = jnp.mean((u - mu) ** 2, axis=0, keepdims=True)
    hn = (u - mu) * lax.rsqrt(var + _EPS) * gam_ref[...] + bet_ref[...]
    h_out_ref[...] = jnp.maximum(hn, 0.0)


def _tc_final_body(msgp_ref, h_ref, wrel_ref, wroot_ref, b_ref, out_ref):
    msg = msgp_ref[pl.ds(0, _N), :] + msgp_ref[pl.ds(_NP, _N), :]
    u = (_dot(msg, wrel_ref[...]) +
         _dot(h_ref[...], wroot_ref[...]) + b_ref[...])
    m = jnp.max(u, axis=-1, keepdims=True)
    lse = jnp.log(jnp.sum(jnp.exp(u - m), axis=-1, keepdims=True)) + m
    out_ref[...] = u - lse


def _tc_pre(x, w_t, d_out):
    return pl.pallas_call(
        _tc_pre_body,
        out_shape=jax.ShapeDtypeStruct((_N, d_out), jnp.float32),
    )(x, w_t)


def _tc_mid(msgp, h, wroot_t, b, gam, bet, wnext_t, d_next):
    return pl.pallas_call(
        _tc_mid_body,
        out_shape=(jax.ShapeDtypeStruct((_N, _D_H), jnp.float32),
                   jax.ShapeDtypeStruct((_N, d_next), jnp.float32)),
    )(msgp, h, wroot_t, b, gam, bet, wnext_t)


def _tc_mid2(msgp, h, wroot_t, b, gam, bet):
    return pl.pallas_call(
        _tc_mid2_body,
        out_shape=jax.ShapeDtypeStruct((_N, _D_H), jnp.float32),
    )(msgp, h, wroot_t, b, gam, bet)


def _tc_final(msgp, h, wrel_t, wroot_t, b):
    return pl.pallas_call(
        _tc_final_body,
        out_shape=jax.ShapeDtypeStruct((_N, _D_OUT), jnp.float32),
    )(msgp, h, wrel_t, wroot_t, b)


def kernel(x, edge_index, W_rel1, b_rel1, W_root1, gamma1, beta1,
           W_rel2, b_rel2, W_root2, gamma2, beta2,
           W_rel3, b_rel3, W_root3):
    src = edge_index[0].reshape(_NC, _NS, _EDGES_PER_TILE)
    dst = edge_index[1].reshape(_NC, _NS, _NCHUNK, _CH)
    zeros_h = jnp.zeros((_CH, _D_H), jnp.float32)

    g1 = _tc_pre(x, W_rel1.T, _D_H)
    m1 = _sc_segment_sum(g1, src, dst, zeros_h, d=_D_H)
    h1, g2 = _tc_mid(m1, x, W_root1.T, b_rel1.reshape(1, -1),
                     gamma1.reshape(1, -1), beta1.reshape(1, -1),
                     W_rel2.T, _D_H)
    m2 = _sc_segment_sum(g2, src, dst, zeros_h, d=_D_H)
    h2 = _tc_mid2(m2, h1, W_root2.T, b_rel2.reshape(1, -1),
                  gamma2.reshape(1, -1), beta2.reshape(1, -1))
    m3 = _sc_segment_sum(h2, src, dst, zeros_h, d=_D_H)
    return _tc_final(m3, h2, W_rel3.T, W_root3.T, b_rel3.reshape(1, -1))
